# baseline (speedup 1.0000x reference)
import functools

import jax
import jax.numpy as jnp
from jax import lax
from jax.experimental import pallas as pl
from jax.experimental.pallas import tpu as pltpu
from jax.experimental.pallas import tpu_sc as plsc

_N = 10000
_E = 160000
_H = 128
_EOUT = 16
_T = 4
_DEG = _E / _N

_NC = 2
_NS = 16
_NW = _NC * _NS
_CH = 64
_EROWS = _E // _CH
_KIT = 80
_NACC = 10008
_NPT = 632
_NPT_LAST = _NACC - 15 * _NPT
_DUMMY = 10000

_f32 = jnp.float32


def _sds(shape):
    return jax.ShapeDtypeStruct(shape, _f32)


def _sc_mesh():
    return plsc.VectorSubcoreMesh(core_axis_name="c", subcore_axis_name="s")


def _elu16(v):
    return jnp.maximum(v, 0.0) + jnp.exp(jnp.minimum(v, 0.0)) - 1.0


def _sc_edge_pass(dec, want_he):
    def body(*args):
        if dec:
            (pq_hbm, g_hbm, cidx_hbm, z_hbm, agg_hbm,
             cbuf, gv, sscat, pqbuf, obuf,
             acc, si0, si1, sg0, sg1) = args
            he_hbm = None
        else:
            (pq_hbm, cidx_hbm, z_hbm, he_hbm, agg_hbm,
             cbuf, sscat, pqbuf, obuf,
             acc, si0, si1, sg0, sg1) = args
            g_hbm = gv = None
        si = (si0, si1)
        sg = (sg0, sg1)

        cid = lax.axis_index("c")
        sid = lax.axis_index("s")
        wid = cid * _NS + sid

        def rowk(k):
            return wid + _NW * k

        def stage(k, b, issue):
            pairs = [(cidx_hbm.at[rowk(k)], cbuf.at[b])]
            if dec:
                pairs.append((g_hbm.at[rowk(k)], gv.at[b]))
            for src, dst in pairs:
                if issue:
                    pltpu.async_copy(src, dst, si[b])
                else:
                    pltpu.make_async_copy(src, dst, si[b]).wait()

        def gat(k, b, issue):
            if issue:
                pltpu.async_copy(pq_hbm.at[cbuf.at[b]], pqbuf.at[b], sg[b])
            else:
                pltpu.make_async_copy(pq_hbm.at[cbuf.at[b]], pqbuf.at[b],
                                      sg[b]).wait()

        def snap(b):
            for j in range(_CH // 16):
                sl = pl.ds(j * 16, 16)
                sscat[pl.ds(b * _CH + j * 16, 16)] = cbuf[b, sl]

        def compute(b):
            base = b * _CH
            if not dec:
                def edge(e, c):
                    for j in range(_H // 16):
                        sl = pl.ds(j * 16, 16)
                        obuf[base + e, sl] = _elu16(
                            pqbuf[b, e, sl] + pqbuf[b, _CH + e, sl])
                    return c
                lax.fori_loop(0, _CH, edge, 0, unroll=2)
            else:
                def group(go, c):
                    g16 = gv[b, pl.ds(go * 16, 16)]
                    for i in range(16):
                        e = go * 16 + i
                        g = g16[i]
                        for j in range(_H // 16):
                            sl = pl.ds(j * 16, 16)
                            obuf[base + e, sl] = _elu16(
                                pqbuf[b, e, sl] + pqbuf[b, _CH + e, sl]) * g
                    return c
                lax.fori_loop(0, _CH // 16, group, 0, unroll=2)

        @pl.when(sid < 15)
        def _():
            sl_own = pl.ds(sid * _NPT, _NPT)
            pltpu.sync_copy(z_hbm.at[sl_own], acc.at[sl_own])

        @pl.when(sid == 15)
        def _():
            sl_last = pl.ds(15 * _NPT, _NPT_LAST)
            pltpu.sync_copy(z_hbm.at[sl_last], acc.at[sl_last])

        stage(0, 0, True)
        stage(1, 1, True)
        plsc.subcore_barrier()
        stage(0, 0, False)
        gat(0, 0, True)

        def pairstep(p2, c):
            c0 = 2 * p2
            c1 = c0 + 1

            @pl.when(rowk(c0) < _EROWS)
            def _():
                gat(c0, 0, False)
                snap(0)

                @pl.when(rowk(c1) < _EROWS)
                def _():
                    stage(c1, 1, False)
                    gat(c1, 1, True)
                compute(0)

                @pl.when(rowk(c0 + 2) < _EROWS)
                def _():
                    stage(c0 + 2, 0, True)
                if want_he:
                    pltpu.sync_copy(obuf.at[pl.ds(0, _CH)],
                                    he_hbm.at[pl.ds(rowk(c0) * _CH, _CH)])

                @pl.when(rowk(c1) < _EROWS)
                def _():
                    gat(c1, 1, False)
                    snap(1)

                    @pl.when(rowk(c1 + 1) < _EROWS)
                    def _():
                        stage(c1 + 1, 0, False)
                        gat(c1 + 1, 0, True)
                    compute(1)

                    @pl.when(rowk(c1 + 2) < _EROWS)
                    def _():
                        stage(c1 + 2, 1, True)
                    if want_he:
                        pltpu.sync_copy(
                            obuf.at[pl.ds(_CH, _CH)],
                            he_hbm.at[pl.ds(rowk(c1) * _CH, _CH)])

                @pl.when(rowk(c1) >= _EROWS)
                def _():
                    dummy = jnp.full((16,), _DUMMY, jnp.int32)
                    for j in range(_CH // 16):
                        sscat[pl.ds(_CH + j * 16, 16)] = dummy

                pltpu.sync_copy(obuf, acc.at[sscat], add=True)
            return c
        lax.fori_loop(0, _KIT // 2, pairstep, 0)

        plsc.subcore_barrier()

        @pl.when(sid < 15)
        def _():
            sl_own = pl.ds(sid * _NPT, _NPT)
            pltpu.sync_copy(acc.at[sl_own], agg_hbm.at[cid, sl_own])

        @pl.when(sid == 15)
        def _():
            sl_last = pl.ds(15 * _NPT, _NPT_LAST)
            pltpu.sync_copy(acc.at[sl_last], agg_hbm.at[cid, sl_last])

    return body


def _sc_scratch(dec):
    sc = [pltpu.VMEM((2, 2 * _CH), jnp.int32)]
    if dec:
        sc.append(pltpu.VMEM((2, _CH), _f32))
    sc += [pltpu.VMEM((2 * _CH,), jnp.int32),
           pltpu.VMEM((2, 2 * _CH, _H), _f32),
           pltpu.VMEM((2 * _CH, _H), _f32),
           pltpu.VMEM_SHARED((_NACC, _H), _f32),
           pltpu.SemaphoreType.DMA,
           pltpu.SemaphoreType.DMA,
           pltpu.SemaphoreType.DMA,
           pltpu.SemaphoreType.DMA]
    return sc


def _sc_edge_encode(pq, cidx2d, zeros):
    return pl.kernel(
        _sc_edge_pass(dec=False, want_he=True),
        out_type=(_sds((_E, _H)), _sds((_NC, _NACC, _H))),
        mesh=_sc_mesh(),
        scratch_types=_sc_scratch(dec=False),
    )(pq, cidx2d, zeros)


def _sc_edge_decode(ab, gate2d, cidx2d, zeros):
    return pl.kernel(
        _sc_edge_pass(dec=True, want_he=False),
        out_type=_sds((_NC, _NACC, _H)),
        mesh=_sc_mesh(),
        scratch_types=_sc_scratch(dec=True),
    )(ab, gate2d, cidx2d, zeros)


_BN = 1000
_BE = 2000


def _mm(a, b):
    return jnp.dot(a, b, preferred_element_type=_f32)


def _elu(v):
    return jnp.where(v > 0.0, v, jnp.exp(jnp.minimum(v, 0.0)) - 1.0)


def _full(shape):
    return pl.BlockSpec(shape, lambda i: tuple(0 for _ in shape))


def _rows(shape):
    return pl.BlockSpec(shape, lambda i: (i,) + tuple(0 for _ in shape[1:]))


def _tc_encode(x, w1, b1, w2, b2, wea, web, be1):
    def body(x_r, w1_r, b1_r, w2_r, b2_r, wea_r, web_r, be1_r,
             h_r, pq_r):
        h1 = _elu(_mm(x_r[...], w1_r[...]) + b1_r[...])
        h = _mm(h1, w2_r[...]) + b2_r[...]
        h_r[...] = h
        pq_r[0] = _mm(h, wea_r[...]) + be1_r[...]
        pq_r[1] = _mm(h, web_r[...])

    return pl.pallas_call(
        body,
        grid=(_N // _BN,),
        in_specs=[_rows((_BN, _H)), _full((_H, _H)), _full((1, _H)),
                  _full((_H, _H)), _full((1, _H)), _full((_H, _H)),
                  _full((_H, _H)), _full((1, _H))],
        out_specs=[_rows((_BN, _H)),
                   pl.BlockSpec((_NC, _BN, _H), lambda i: (0, i, 0))],
        out_shape=(_sds((_N, _H)), _sds((_NC, _N, _H))),
    )(x, w1, b1, w2, b2, wea, web, be1)


def _tc_edge_weight(he, we2, be2, wg, bg):
    def body(he_r, we2_r, be2_r, wg_r, bg_r, ew_r, g_r):
        ew = _mm(he_r[...], we2_r[...]) + be2_r[...]
        ew_r[...] = ew
        g_r[...] = jax.nn.sigmoid(
            jnp.sum(ew * wg_r[...], axis=1, keepdims=True) + bg_r[...])

    return pl.pallas_call(
        body,
        grid=(_E // _BE,),
        in_specs=[_rows((_BE, _H)), _full((_H, _EOUT)), _full((1, _EOUT)),
                  _full((1, _EOUT)), _full((1, 1))],
        out_specs=[_rows((_BE, _EOUT)), _rows((_BE, 1))],
        out_shape=(_sds((_E, _EOUT)), _sds((_E, 1))),
    )(he, we2, be2, wg, bg)


def _tc_node0(h, aggp, wna, wnb, bn1, wma, wmb, bm):
    def body(h_r, ag_r, wna_r, wnb_r, bn1_r, wma_r, wmb_r, bm_r,
             hs_r, ab_r):
        agg = (ag_r[0] + ag_r[1]) * (1.0 / _DEG)
        hs = _elu(_mm(h_r[...], wna_r[...]) + _mm(agg, wnb_r[...]) + bn1_r[...])
        hs_r[...] = hs
        ab_r[0] = _mm(hs, wma_r[...]) + bm_r[...]
        ab_r[1] = _mm(hs, wmb_r[...])

    return pl.pallas_call(
        body,
        grid=(_N // _BN,),
        in_specs=[_rows((_BN, _H)),
                  pl.BlockSpec((_NC, _BN, _H), lambda i: (0, i, 0)),
                  _full((_H, _H)), _full((_H, _H)), _full((1, _H)),
                  _full((_H, _H)), _full((_H, _H)), _full((1, _H))],
        out_specs=[_rows((_BN, _H)),
                   pl.BlockSpec((_NC, _BN, _H), lambda i: (0, i, 0))],
        out_shape=(_sds((_N, _H)), _sds((_NC, _N, _H))),
    )(h, aggp, wna, wnb, bn1, wma, wmb, bm)


def _tc_gru(dec, maggp, hstate, wia, wib, bi, wh, bh, wout, bout,
            wma, wmb, bm):
    def body(d_r, mg_r, h_r, wia_r, wib_r, bi_r, wh_r, bh_r,
             wout_r, bout_r, wma_r, wmb_r, bm_r,
             hn_r, mean_r, ab_r):
        magg = (mg_r[0] + mg_r[1]) * (1.0 / _DEG)
        h = h_r[...]
        gi = _mm(d_r[...], wia_r[...]) + _mm(magg, wib_r[...]) + bi_r[...]
        gh = _mm(h, wh_r[...]) + bh_r[...]
        r = jax.nn.sigmoid(gi[:, 0:_H] + gh[:, 0:_H])
        z = jax.nn.sigmoid(gi[:, _H:2 * _H] + gh[:, _H:2 * _H])
        n = jnp.tanh(gi[:, 2 * _H:3 * _H] + r * gh[:, 2 * _H:3 * _H])
        hn = (1.0 - z) * h + z * n
        hn_r[...] = hn
        mean_r[...] = _mm(hn, wout_r[...]) + bout_r[...]
        ab_r[0] = _mm(hn, wma_r[...]) + bm_r[...]
        ab_r[1] = _mm(hn, wmb_r[...])

    return pl.pallas_call(
        body,
        grid=(_N // _BN,),
        in_specs=[_rows((_BN, _H)),
                  pl.BlockSpec((_NC, _BN, _H), lambda i: (0, i, 0)),
                  _rows((_BN, _H)),
                  _full((_H, 3 * _H)), _full((_H, 3 * _H)), _full((1, 3 * _H)),
                  _full((_H, 3 * _H)), _full((1, 3 * _H)),
                  _full((_H, _H)), _full((1, _H)),
                  _full((_H, _H)), _full((_H, _H)), _full((1, _H))],
        out_specs=[_rows((_BN, _H)), _rows((_BN, _H)),
                   pl.BlockSpec((_NC, _BN, _H), lambda i: (0, i, 0))],
        out_shape=(_sds((_N, _H)), _sds((_N, _H)), _sds((_NC, _N, _H))),
    )(dec, maggp, hstate, wia, wib, bi, wh, bh, wout, bout, wma, wmb, bm)


def kernel(x, decoder_input, recv_idx, send_idx, W1, b1, W2, b2, We1, be1,
           We2, be2, Wn1, bn1, Wm, bm, Wg, bg, Wi, bi, Wh, bh, Wout, bout):
    cidx2d = jnp.concatenate([recv_idx.reshape(_EROWS, _CH),
                              send_idx.reshape(_EROWS, _CH) + _N], axis=1)
    zeros = jnp.zeros((_NACC, _H), _f32)

    row = lambda v: v.reshape(1, -1)

    h, pq = _tc_encode(x, W1, row(b1), W2, row(b2),
                       We1[:_H], We1[_H:], row(be1))

    he, aggp = _sc_edge_encode(pq.reshape(2 * _N, _H), cidx2d, zeros)

    edge_weight, gate = _tc_edge_weight(he, We2, row(be2),
                                        Wg.reshape(1, _EOUT), bg.reshape(1, 1))
    gate2d = gate.reshape(_EROWS, _CH)

    hstate, ab = _tc_node0(h, aggp, Wn1[:_H], Wn1[_H:], row(bn1),
                           Wm[:_H], Wm[_H:], row(bm))

    means = []
    for t in range(_T):
        maggp = _sc_edge_decode(ab.reshape(2 * _N, _H), gate2d, cidx2d, zeros)
        hstate, mean, ab = _tc_gru(
            decoder_input[t], maggp, hstate,
            Wi[:_H], Wi[_H:], row(bi), Wh, row(bh),
            Wout, row(bout), Wm[:_H], Wm[_H:], row(bm))
        means.append(mean)

    return jnp.stack(means, axis=0), edge_weight


# speedup vs baseline: 1.5725x; 1.5725x over previous
import functools

import jax
import jax.numpy as jnp
from jax import lax
from jax.experimental import pallas as pl
from jax.experimental.pallas import tpu as pltpu
from jax.experimental.pallas import tpu_sc as plsc

_N = 10000
_E = 160000
_H = 128
_EOUT = 16
_T = 4
_DEG = _E / _N

_NC = 2
_NS = 16
_NW = _NC * _NS
_CH = 64
_EROWS = _E // _CH
_KIT = 80
_NACC = 10008
_NPT = 632
_NPT_LAST = _NACC - 15 * _NPT
_DUMMY = 10000

_f32 = jnp.float32


def _sds(shape):
    return jax.ShapeDtypeStruct(shape, _f32)


def _sc_mesh():
    return plsc.VectorSubcoreMesh(core_axis_name="c", subcore_axis_name="s")


def _elu16(v):
    return jnp.maximum(v, 0.0) + jnp.exp(jnp.minimum(v, 0.0)) - 1.0


def _sc_edge_pass(dec, want_he):
    def body(*args):
        if dec:
            (pq_hbm, g_hbm, cidx_hbm, z_hbm, agg_hbm,
             cbuf, gv, sscat, pqbuf, obuf,
             acc, si0, si1, sg0, sg1) = args
            he_hbm = None
        else:
            (pq_hbm, cidx_hbm, z_hbm, he_hbm, agg_hbm,
             cbuf, sscat, pqbuf, obuf,
             acc, si0, si1, sg0, sg1) = args
            g_hbm = gv = None
        si = (si0, si1)
        sg = (sg0, sg1)

        cid = lax.axis_index("c")
        sid = lax.axis_index("s")
        wid = cid * _NS + sid

        def rowk(k):
            return wid + _NW * k

        def stage(k, b, issue):
            pairs = [(cidx_hbm.at[rowk(k)], cbuf.at[b])]
            if dec:
                pairs.append((g_hbm.at[rowk(k)], gv.at[b]))
            for src, dst in pairs:
                if issue:
                    pltpu.async_copy(src, dst, si[b])
                else:
                    pltpu.make_async_copy(src, dst, si[b]).wait()

        def gat(k, b, issue):
            if issue:
                pltpu.async_copy(pq_hbm.at[cbuf.at[b]], pqbuf.at[b], sg[b])
            else:
                pltpu.make_async_copy(pq_hbm.at[cbuf.at[b]], pqbuf.at[b],
                                      sg[b]).wait()

        def snap(b):
            for j in range(_CH // 16):
                sl = pl.ds(j * 16, 16)
                sscat[pl.ds(b * _CH + j * 16, 16)] = cbuf[b, sl]

        def compute(b):
            base = b * _CH
            if not dec:
                def edge(e, c):
                    for j in range(_H // 16):
                        sl = pl.ds(j * 16, 16)
                        obuf[base + e, sl] = _elu16(
                            pqbuf[b, e, sl] + pqbuf[b, _CH + e, sl])
                    return c
                lax.fori_loop(0, _CH, edge, 0)
            else:
                def group(go, c):
                    g16 = gv[b, pl.ds(go * 16, 16)]
                    for i in range(16):
                        e = go * 16 + i
                        g = g16[i]
                        for j in range(_H // 16):
                            sl = pl.ds(j * 16, 16)
                            obuf[base + e, sl] = _elu16(
                                pqbuf[b, e, sl] + pqbuf[b, _CH + e, sl]) * g
                    return c
                lax.fori_loop(0, _CH // 16, group, 0)

        @pl.when(sid < 15)
        def _():
            sl_own = pl.ds(sid * _NPT, _NPT)
            pltpu.sync_copy(z_hbm.at[sl_own], acc.at[sl_own])

        @pl.when(sid == 15)
        def _():
            sl_last = pl.ds(15 * _NPT, _NPT_LAST)
            pltpu.sync_copy(z_hbm.at[sl_last], acc.at[sl_last])

        stage(0, 0, True)
        stage(1, 1, True)
        plsc.subcore_barrier()
        stage(0, 0, False)
        gat(0, 0, True)

        def pairstep(p2, c):
            c0 = 2 * p2
            c1 = c0 + 1

            @pl.when(rowk(c0) < _EROWS)
            def _():
                gat(c0, 0, False)
                snap(0)

                @pl.when(rowk(c1) < _EROWS)
                def _():
                    stage(c1, 1, False)
                    gat(c1, 1, True)
                compute(0)

                @pl.when(rowk(c0 + 2) < _EROWS)
                def _():
                    stage(c0 + 2, 0, True)
                if want_he:
                    pltpu.sync_copy(obuf.at[pl.ds(0, _CH)],
                                    he_hbm.at[pl.ds(rowk(c0) * _CH, _CH)])

                @pl.when(rowk(c1) < _EROWS)
                def _():
                    gat(c1, 1, False)
                    snap(1)

                    @pl.when(rowk(c1 + 1) < _EROWS)
                    def _():
                        stage(c1 + 1, 0, False)
                        gat(c1 + 1, 0, True)
                    compute(1)

                    @pl.when(rowk(c1 + 2) < _EROWS)
                    def _():
                        stage(c1 + 2, 1, True)
                    if want_he:
                        pltpu.sync_copy(
                            obuf.at[pl.ds(_CH, _CH)],
                            he_hbm.at[pl.ds(rowk(c1) * _CH, _CH)])

                @pl.when(rowk(c1) >= _EROWS)
                def _():
                    dummy = jnp.full((16,), _DUMMY, jnp.int32)
                    for j in range(_CH // 16):
                        sscat[pl.ds(_CH + j * 16, 16)] = dummy

                pltpu.sync_copy(obuf, acc.at[sscat], add=True)
            return c
        lax.fori_loop(0, _KIT // 2, pairstep, 0)

        plsc.subcore_barrier()

        @pl.when(sid < 15)
        def _():
            sl_own = pl.ds(sid * _NPT, _NPT)
            pltpu.sync_copy(acc.at[sl_own], agg_hbm.at[cid, sl_own])

        @pl.when(sid == 15)
        def _():
            sl_last = pl.ds(15 * _NPT, _NPT_LAST)
            pltpu.sync_copy(acc.at[sl_last], agg_hbm.at[cid, sl_last])

    return body


def _sc_scratch(dec):
    sc = [pltpu.VMEM((2, 2 * _CH), jnp.int32)]
    if dec:
        sc.append(pltpu.VMEM((2, _CH), _f32))
    sc += [pltpu.VMEM((2 * _CH,), jnp.int32),
           pltpu.VMEM((2, 2 * _CH, _H), _f32),
           pltpu.VMEM((2 * _CH, _H), _f32),
           pltpu.VMEM_SHARED((_NACC, _H), _f32),
           pltpu.SemaphoreType.DMA,
           pltpu.SemaphoreType.DMA,
           pltpu.SemaphoreType.DMA,
           pltpu.SemaphoreType.DMA]
    return sc


def _sc_edge_encode(pq, cidx2d, zeros):
    return pl.kernel(
        _sc_edge_pass(dec=False, want_he=True),
        out_type=(_sds((_E, _H)), _sds((_NC, _NACC, _H))),
        mesh=_sc_mesh(),
        scratch_types=_sc_scratch(dec=False),
    )(pq, cidx2d, zeros)


def _sc_edge_decode(ab, gate2d, cidx2d, zeros):
    return pl.kernel(
        _sc_edge_pass(dec=True, want_he=False),
        out_type=_sds((_NC, _NACC, _H)),
        mesh=_sc_mesh(),
        scratch_types=_sc_scratch(dec=True),
    )(ab, gate2d, cidx2d, zeros)


_BN = 1000
_BE = 2000


def _mm(a, b):
    return jnp.dot(a, b, preferred_element_type=_f32)


def _elu(v):
    return jnp.where(v > 0.0, v, jnp.exp(jnp.minimum(v, 0.0)) - 1.0)


def _full(shape):
    return pl.BlockSpec(shape, lambda i: tuple(0 for _ in shape))


def _rows(shape):
    return pl.BlockSpec(shape, lambda i: (i,) + tuple(0 for _ in shape[1:]))


def _tc_encode(x, w1, b1, w2, b2, wea, web, be1):
    def body(x_r, w1_r, b1_r, w2_r, b2_r, wea_r, web_r, be1_r,
             h_r, pq_r):
        h1 = _elu(_mm(x_r[...], w1_r[...]) + b1_r[...])
        h = _mm(h1, w2_r[...]) + b2_r[...]
        h_r[...] = h
        pq_r[0] = _mm(h, wea_r[...]) + be1_r[...]
        pq_r[1] = _mm(h, web_r[...])

    return pl.pallas_call(
        body,
        grid=(_N // _BN,),
        in_specs=[_rows((_BN, _H)), _full((_H, _H)), _full((1, _H)),
                  _full((_H, _H)), _full((1, _H)), _full((_H, _H)),
                  _full((_H, _H)), _full((1, _H))],
        out_specs=[_rows((_BN, _H)),
                   pl.BlockSpec((_NC, _BN, _H), lambda i: (0, i, 0))],
        out_shape=(_sds((_N, _H)), _sds((_NC, _N, _H))),
    )(x, w1, b1, w2, b2, wea, web, be1)


def _tc_edge_weight(he, we2, be2, wg, bg):
    def body(he_r, we2_r, be2_r, wg_r, bg_r, ew_r, g_r):
        ew = _mm(he_r[...], we2_r[...]) + be2_r[...]
        ew_r[...] = ew
        g_r[...] = jax.nn.sigmoid(
            jnp.sum(ew * wg_r[...], axis=1, keepdims=True) + bg_r[...])

    return pl.pallas_call(
        body,
        grid=(_E // _BE,),
        in_specs=[_rows((_BE, _H)), _full((_H, _EOUT)), _full((1, _EOUT)),
                  _full((1, _EOUT)), _full((1, 1))],
        out_specs=[_rows((_BE, _EOUT)), _rows((_BE, 1))],
        out_shape=(_sds((_E, _EOUT)), _sds((_E, 1))),
    )(he, we2, be2, wg, bg)


def _tc_node0(h, aggp, wna, wnb, bn1, wma, wmb, bm):
    def body(h_r, ag_r, wna_r, wnb_r, bn1_r, wma_r, wmb_r, bm_r,
             hs_r, ab_r):
        agg = (ag_r[0] + ag_r[1]) * (1.0 / _DEG)
        hs = _elu(_mm(h_r[...], wna_r[...]) + _mm(agg, wnb_r[...]) + bn1_r[...])
        hs_r[...] = hs
        ab_r[0] = _mm(hs, wma_r[...]) + bm_r[...]
        ab_r[1] = _mm(hs, wmb_r[...])

    return pl.pallas_call(
        body,
        grid=(_N // _BN,),
        in_specs=[_rows((_BN, _H)),
                  pl.BlockSpec((_NC, _BN, _H), lambda i: (0, i, 0)),
                  _full((_H, _H)), _full((_H, _H)), _full((1, _H)),
                  _full((_H, _H)), _full((_H, _H)), _full((1, _H))],
        out_specs=[_rows((_BN, _H)),
                   pl.BlockSpec((_NC, _BN, _H), lambda i: (0, i, 0))],
        out_shape=(_sds((_N, _H)), _sds((_NC, _N, _H))),
    )(h, aggp, wna, wnb, bn1, wma, wmb, bm)


def _tc_gru(dec, maggp, hstate, wia, wib, bi, wh, bh, wout, bout,
            wma, wmb, bm):
    def body(d_r, mg_r, h_r, wia_r, wib_r, bi_r, wh_r, bh_r,
             wout_r, bout_r, wma_r, wmb_r, bm_r,
             hn_r, mean_r, ab_r):
        magg = (mg_r[0] + mg_r[1]) * (1.0 / _DEG)
        h = h_r[...]
        gi = _mm(d_r[...], wia_r[...]) + _mm(magg, wib_r[...]) + bi_r[...]
        gh = _mm(h, wh_r[...]) + bh_r[...]
        r = jax.nn.sigmoid(gi[:, 0:_H] + gh[:, 0:_H])
        z = jax.nn.sigmoid(gi[:, _H:2 * _H] + gh[:, _H:2 * _H])
        n = jnp.tanh(gi[:, 2 * _H:3 * _H] + r * gh[:, 2 * _H:3 * _H])
        hn = (1.0 - z) * h + z * n
        hn_r[...] = hn
        mean_r[...] = _mm(hn, wout_r[...]) + bout_r[...]
        ab_r[0] = _mm(hn, wma_r[...]) + bm_r[...]
        ab_r[1] = _mm(hn, wmb_r[...])

    return pl.pallas_call(
        body,
        grid=(_N // _BN,),
        in_specs=[_rows((_BN, _H)),
                  pl.BlockSpec((_NC, _BN, _H), lambda i: (0, i, 0)),
                  _rows((_BN, _H)),
                  _full((_H, 3 * _H)), _full((_H, 3 * _H)), _full((1, 3 * _H)),
                  _full((_H, 3 * _H)), _full((1, 3 * _H)),
                  _full((_H, _H)), _full((1, _H)),
                  _full((_H, _H)), _full((_H, _H)), _full((1, _H))],
        out_specs=[_rows((_BN, _H)), _rows((_BN, _H)),
                   pl.BlockSpec((_NC, _BN, _H), lambda i: (0, i, 0))],
        out_shape=(_sds((_N, _H)), _sds((_N, _H)), _sds((_NC, _N, _H))),
    )(dec, maggp, hstate, wia, wib, bi, wh, bh, wout, bout, wma, wmb, bm)


def kernel(x, decoder_input, recv_idx, send_idx, W1, b1, W2, b2, We1, be1,
           We2, be2, Wn1, bn1, Wm, bm, Wg, bg, Wi, bi, Wh, bh, Wout, bout):
    cidx2d = jnp.concatenate([recv_idx.reshape(_EROWS, _CH),
                              send_idx.reshape(_EROWS, _CH) + _N], axis=1)
    zeros = jnp.zeros((_NACC, _H), _f32)

    row = lambda v: v.reshape(1, -1)

    h, pq = _tc_encode(x, W1, row(b1), W2, row(b2),
                       We1[:_H], We1[_H:], row(be1))

    he, aggp = _sc_edge_encode(pq.reshape(2 * _N, _H), cidx2d, zeros)

    edge_weight, gate = _tc_edge_weight(he, We2, row(be2),
                                        Wg.reshape(1, _EOUT), bg.reshape(1, 1))
    gate2d = gate.reshape(_EROWS, _CH)

    hstate, ab = _tc_node0(h, aggp, Wn1[:_H], Wn1[_H:], row(bn1),
                           Wm[:_H], Wm[_H:], row(bm))

    means = []
    for t in range(_T):
        maggp = _sc_edge_decode(ab.reshape(2 * _N, _H), gate2d, cidx2d, zeros)
        hstate, mean, ab = _tc_gru(
            decoder_input[t], maggp, hstate,
            Wi[:_H], Wi[_H:], row(bi), Wh, row(bh),
            Wout, row(bout), Wm[:_H], Wm[_H:], row(bm))
        means.append(mean)

    return jnp.stack(means, axis=0), edge_weight
